# baseline (speedup 1.0000x reference)
import functools

import jax
import jax.numpy as jnp
from jax import lax
from jax.experimental import pallas as pl
from jax.experimental.pallas import tpu as pltpu
from jax.experimental.pallas import tpu_sc as plsc

N = 10000
E = 320000
D = 128
M = 256

NC = 2
NS = 16
NW = NC * NS

N_PAD = 10240
ROWS_PER_TILE = N_PAD // NS
CHUNK = 64
E_PAD = 327680
CHUNKS_PER_TILE = E_PAD // (NW * CHUNK)
PHASES = 4
PCH = CHUNKS_PER_TILE // PHASES

_mesh = plsc.VectorSubcoreMesh(core_axis_name="c", subcore_axis_name="s")


DEG_DEPTH = 8


@functools.partial(
    pl.kernel,
    out_type=jax.ShapeDtypeStruct((NC, N_PAD, D), jnp.float32),
    mesh=_mesh,
    scratch_types=[
        pltpu.VMEM((CHUNKS_PER_TILE, CHUNK), jnp.int32),
        pltpu.VMEM((CHUNK, D), jnp.float32),
        pltpu.VMEM_SHARED((N_PAD, D), jnp.float32),
        pltpu.SemaphoreType.DMA,
    ],
)
def _deg_kernel(dstp_hbm, zrows_hbm, ones_hbm, out_hbm, dst_v, ones_v, deg_acc,
                dsem):
    c = lax.axis_index("c")
    s = lax.axis_index("s")
    wid = c * NS + s
    pltpu.sync_copy(zrows_hbm, deg_acc.at[pl.ds(s * ROWS_PER_TILE, ROWS_PER_TILE)])
    pltpu.sync_copy(ones_hbm, ones_v)
    pltpu.sync_copy(dstp_hbm.at[pl.ds(wid * CHUNKS_PER_TILE, CHUNKS_PER_TILE)], dst_v)
    plsc.subcore_barrier()

    def body(j, carry):
        @pl.when(j >= DEG_DEPTH)
        def _():
            pltpu.make_async_copy(ones_v, deg_acc.at[dst_v.at[0]], dsem).wait()

        pltpu.async_copy(ones_v, deg_acc.at[dst_v.at[j]], dsem, add=True)
        return carry

    lax.fori_loop(0, CHUNKS_PER_TILE, body, 0)

    def drain(j, carry):
        pltpu.make_async_copy(ones_v, deg_acc.at[dst_v.at[0]], dsem).wait()
        return carry

    lax.fori_loop(0, DEG_DEPTH, drain, 0)
    plsc.subcore_barrier()
    pltpu.sync_copy(
        deg_acc.at[pl.ds(s * ROWS_PER_TILE, ROWS_PER_TILE)],
        out_hbm.at[c].at[pl.ds(s * ROWS_PER_TILE, ROWS_PER_TILE)],
    )


NBUF = 4
LOOKA = 2
CA = 320
CB = 0
NPH_MAX = max(CA, CB) // PCH


@functools.partial(
    pl.kernel,
    out_type=jax.ShapeDtypeStruct((NC, N_PAD, D), jnp.float32),
    mesh=_mesh,
    scratch_types=[
        pltpu.VMEM((PCH, CHUNK), jnp.int32),
        pltpu.VMEM((PCH, CHUNK), jnp.int32),
        [pltpu.VMEM((CHUNK, D), jnp.float32)] * NBUF,
        pltpu.VMEM_SHARED((N_PAD, D), jnp.float32),
        [pltpu.SemaphoreType.DMA] * NBUF,
        [pltpu.SemaphoreType.DMA] * NBUF,
    ],
)
def _scatter_kernel(y_hbm, srcp_hbm, dstp_hbm, zrows_hbm, out_hbm,
                    src_v, dst_v, rows, acc, gsem, ssem):
    c = lax.axis_index("c")
    s = lax.axis_index("s")
    cnt = lax.select(c == 0, CA, CB)
    base = c * (NS * CA) + s * cnt
    pltpu.sync_copy(zrows_hbm, acc.at[pl.ds(s * ROWS_PER_TILE, ROWS_PER_TILE)])
    plsc.subcore_barrier()

    for p in range(NPH_MAX):

        @pl.when(p * PCH < cnt)
        def _():
            pltpu.sync_copy(srcp_hbm.at[pl.ds(base + p * PCH, PCH)], src_v)
            pltpu.sync_copy(dstp_hbm.at[pl.ds(base + p * PCH, PCH)], dst_v)

            for k in range(LOOKA):
                pltpu.async_copy(y_hbm.at[src_v.at[k]], rows[k % NBUF],
                                 gsem[k % NBUF])

            def outer(g, carry):
                for b in range(NBUF):
                    k = g * NBUF + b
                    pltpu.make_async_copy(
                        y_hbm.at[src_v.at[0]], rows[b], gsem[b]).wait()
                    pltpu.async_copy(rows[b], acc.at[dst_v.at[k]], ssem[b],
                                     add=True)
                    b2 = (b + LOOKA) % NBUF

                    @pl.when(k + LOOKA >= NBUF)
                    def _():
                        pltpu.make_async_copy(
                            rows[b2], acc.at[dst_v.at[0]], ssem[b2]).wait()

                    @pl.when(k + LOOKA < PCH)
                    def _():
                        pltpu.async_copy(
                            y_hbm.at[src_v.at[k + LOOKA]], rows[b2], gsem[b2])
                return carry

            lax.fori_loop(0, PCH // NBUF, outer, 0)

            for k in range(PCH - NBUF + LOOKA, PCH):
                b = k % NBUF
                pltpu.make_async_copy(rows[b], acc.at[dst_v.at[0]],
                                      ssem[b]).wait()

    plsc.subcore_barrier()
    pltpu.sync_copy(
        acc.at[pl.ds(s * ROWS_PER_TILE, ROWS_PER_TILE)],
        out_hbm.at[c].at[pl.ds(s * ROWS_PER_TILE, ROWS_PER_TILE)],
    )


def _y_body(x_ref, w_ref, d0_ref, d1_ref, y_ref):
    deg = d0_ref[:, 0:1] + d1_ref[:, 0:1] + 1.0
    xl = jnp.dot(x_ref[:], w_ref[:], preferred_element_type=jnp.float32)
    y_ref[:] = xl * lax.rsqrt(deg)


def _mlp_body(z0_ref, z1_ref, y_ref, d0_ref, d1_ref, x_ref, bc_ref,
              w1_ref, b1_ref, w2_ref, b2_ref, w3_ref, b3_ref, o_ref):
    deg = d0_ref[:, 0:1] + d1_ref[:, 0:1] + 1.0
    dinv = lax.rsqrt(deg)
    agg = (z0_ref[:] + z1_ref[:] + y_ref[:]) * dinv
    out = jnp.maximum(agg + bc_ref[:], 0.0)
    h = out + x_ref[:]
    h = jnp.maximum(
        jnp.dot(h, w1_ref[:], preferred_element_type=jnp.float32) + b1_ref[:], 0.0)
    h = jnp.maximum(
        jnp.dot(h, w2_ref[:], preferred_element_type=jnp.float32) + b2_ref[:], 0.0)
    o_ref[:] = jnp.dot(h, w3_ref[:], preferred_element_type=jnp.float32) + b3_ref[:]


ROW_BLK = 640


@jax.jit
def kernel(x, edge_index, W_conv, b_conv, W1, b1, W2, b2, W3, b3):
    src = edge_index[0]
    dst = edge_index[1]
    pad = E_PAD - E
    srcp = jnp.concatenate([src, jnp.zeros((pad,), jnp.int32)]).reshape(
        NW * CHUNKS_PER_TILE, CHUNK)
    dstp = jnp.concatenate([dst, jnp.full((pad,), N, jnp.int32)]).reshape(
        NW * CHUNKS_PER_TILE, CHUNK)
    x_pad = jnp.zeros((N_PAD, D), jnp.float32).at[:N].set(x)

    ones_rows = jnp.ones((CHUNK, D), jnp.float32)
    zrows = jnp.zeros((ROWS_PER_TILE, D), jnp.float32)

    deg_parts = _deg_kernel(dstp, zrows, ones_rows)
    deg0, deg1 = deg_parts[0], deg_parts[1]

    y = pl.pallas_call(
        _y_body,
        grid=(N_PAD // ROW_BLK,),
        in_specs=[
            pl.BlockSpec((ROW_BLK, D), lambda i: (i, 0)),
            pl.BlockSpec((D, D), lambda i: (0, 0)),
            pl.BlockSpec((ROW_BLK, D), lambda i: (i, 0)),
            pl.BlockSpec((ROW_BLK, D), lambda i: (i, 0)),
        ],
        out_specs=pl.BlockSpec((ROW_BLK, D), lambda i: (i, 0)),
        out_shape=jax.ShapeDtypeStruct((N_PAD, D), jnp.float32),
    )(x_pad, W_conv, deg0, deg1)

    z_parts = _scatter_kernel(y, srcp, dstp, zrows)
    z0, z1 = z_parts[0], z_parts[1]

    out_pad = pl.pallas_call(
        _mlp_body,
        grid=(N_PAD // ROW_BLK,),
        in_specs=[
            pl.BlockSpec((ROW_BLK, D), lambda i: (i, 0)),
            pl.BlockSpec((ROW_BLK, D), lambda i: (i, 0)),
            pl.BlockSpec((ROW_BLK, D), lambda i: (i, 0)),
            pl.BlockSpec((ROW_BLK, D), lambda i: (i, 0)),
            pl.BlockSpec((ROW_BLK, D), lambda i: (i, 0)),
            pl.BlockSpec((ROW_BLK, D), lambda i: (i, 0)),
            pl.BlockSpec((1, D), lambda i: (0, 0)),
            pl.BlockSpec((D, M), lambda i: (0, 0)),
            pl.BlockSpec((1, M), lambda i: (0, 0)),
            pl.BlockSpec((M, M), lambda i: (0, 0)),
            pl.BlockSpec((1, M), lambda i: (0, 0)),
            pl.BlockSpec((M, 1), lambda i: (0, 0)),
            pl.BlockSpec((1, 1), lambda i: (0, 0)),
        ],
        out_specs=pl.BlockSpec((ROW_BLK, 1), lambda i: (i, 0)),
        out_shape=jax.ShapeDtypeStruct((N_PAD, 1), jnp.float32),
    )(z0, z1, y, deg0, deg1, x_pad,
      b_conv.reshape(1, D), W1, b1.reshape(1, M), W2, b2.reshape(1, M),
      W3, b3.reshape(1, 1))

    return out_pad[:N]


# speedup vs baseline: 1.2108x; 1.2108x over previous
import functools

import jax
import jax.numpy as jnp
from jax import lax
from jax.experimental import pallas as pl
from jax.experimental.pallas import tpu as pltpu
from jax.experimental.pallas import tpu_sc as plsc

N = 10000
E = 320000
D = 128
M = 256

NC = 2
NS = 16
NW = NC * NS

N_PAD = 10240
ROWS_PER_TILE = N_PAD // NS
CHUNK = 64
E_PAD = 327680
CHUNKS_PER_TILE = E_PAD // (NW * CHUNK)
PHASES = 4
PCH = CHUNKS_PER_TILE // PHASES

_mesh = plsc.VectorSubcoreMesh(core_axis_name="c", subcore_axis_name="s")


DEG_DEPTH = 8


@functools.partial(
    pl.kernel,
    out_type=jax.ShapeDtypeStruct((NC, N_PAD, D), jnp.float32),
    mesh=_mesh,
    scratch_types=[
        pltpu.VMEM((CHUNKS_PER_TILE, CHUNK), jnp.int32),
        pltpu.VMEM((CHUNK, D), jnp.float32),
        pltpu.VMEM_SHARED((N_PAD, D), jnp.float32),
        pltpu.SemaphoreType.DMA,
    ],
)
def _deg_kernel(dstp_hbm, zrows_hbm, ones_hbm, out_hbm, dst_v, ones_v, deg_acc,
                dsem):
    c = lax.axis_index("c")
    s = lax.axis_index("s")
    wid = c * NS + s
    pltpu.sync_copy(zrows_hbm, deg_acc.at[pl.ds(s * ROWS_PER_TILE, ROWS_PER_TILE)])
    pltpu.sync_copy(ones_hbm, ones_v)
    pltpu.sync_copy(dstp_hbm.at[pl.ds(wid * CHUNKS_PER_TILE, CHUNKS_PER_TILE)], dst_v)
    plsc.subcore_barrier()

    def body(j, carry):
        @pl.when(j >= DEG_DEPTH)
        def _():
            pltpu.make_async_copy(ones_v, deg_acc.at[dst_v.at[0]], dsem).wait()

        pltpu.async_copy(ones_v, deg_acc.at[dst_v.at[j]], dsem, add=True)
        return carry

    lax.fori_loop(0, CHUNKS_PER_TILE, body, 0)

    def drain(j, carry):
        pltpu.make_async_copy(ones_v, deg_acc.at[dst_v.at[0]], dsem).wait()
        return carry

    lax.fori_loop(0, DEG_DEPTH, drain, 0)
    plsc.subcore_barrier()
    pltpu.sync_copy(
        deg_acc.at[pl.ds(s * ROWS_PER_TILE, ROWS_PER_TILE)],
        out_hbm.at[c].at[pl.ds(s * ROWS_PER_TILE, ROWS_PER_TILE)],
    )


NBUF = 4
LOOKA = 2
CA = 280
CB = 40
NPH_MAX = max(CA, CB) // PCH


@functools.partial(
    pl.kernel,
    out_type=jax.ShapeDtypeStruct((NC, N_PAD, D), jnp.float32),
    mesh=_mesh,
    scratch_types=[
        pltpu.VMEM((PCH, CHUNK), jnp.int32),
        pltpu.VMEM((PCH, CHUNK), jnp.int32),
        [pltpu.VMEM((CHUNK, D), jnp.float32)] * NBUF,
        pltpu.VMEM_SHARED((N_PAD, D), jnp.float32),
        [pltpu.SemaphoreType.DMA] * NBUF,
        [pltpu.SemaphoreType.DMA] * NBUF,
    ],
)
def _scatter_kernel(y_hbm, srcp_hbm, dstp_hbm, zrows_hbm, out_hbm,
                    src_v, dst_v, rows, acc, gsem, ssem):
    c = lax.axis_index("c")
    s = lax.axis_index("s")
    cnt = lax.select(c == 0, CA, CB)
    base = c * (NS * CA) + s * cnt
    pltpu.sync_copy(zrows_hbm, acc.at[pl.ds(s * ROWS_PER_TILE, ROWS_PER_TILE)])
    plsc.subcore_barrier()

    for p in range(NPH_MAX):

        @pl.when(p * PCH < cnt)
        def _():
            pltpu.sync_copy(srcp_hbm.at[pl.ds(base + p * PCH, PCH)], src_v)
            pltpu.sync_copy(dstp_hbm.at[pl.ds(base + p * PCH, PCH)], dst_v)

            for k in range(LOOKA):
                pltpu.async_copy(y_hbm.at[src_v.at[k]], rows[k % NBUF],
                                 gsem[k % NBUF])

            def outer(g, carry):
                for b in range(NBUF):
                    k = g * NBUF + b
                    pltpu.make_async_copy(
                        y_hbm.at[src_v.at[0]], rows[b], gsem[b]).wait()
                    pltpu.async_copy(rows[b], acc.at[dst_v.at[k]], ssem[b],
                                     add=True)
                    b2 = (b + LOOKA) % NBUF

                    @pl.when(k + LOOKA >= NBUF)
                    def _():
                        pltpu.make_async_copy(
                            rows[b2], acc.at[dst_v.at[0]], ssem[b2]).wait()

                    @pl.when(k + LOOKA < PCH)
                    def _():
                        pltpu.async_copy(
                            y_hbm.at[src_v.at[k + LOOKA]], rows[b2], gsem[b2])
                return carry

            lax.fori_loop(0, PCH // NBUF, outer, 0)

            for k in range(PCH - NBUF + LOOKA, PCH):
                b = k % NBUF
                pltpu.make_async_copy(rows[b], acc.at[dst_v.at[0]],
                                      ssem[b]).wait()

    plsc.subcore_barrier()
    pltpu.sync_copy(
        acc.at[pl.ds(s * ROWS_PER_TILE, ROWS_PER_TILE)],
        out_hbm.at[c].at[pl.ds(s * ROWS_PER_TILE, ROWS_PER_TILE)],
    )


def _y_body(x_ref, w_ref, d0_ref, d1_ref, y_ref):
    deg = d0_ref[:, 0:1] + d1_ref[:, 0:1] + 1.0
    xl = jnp.dot(x_ref[:], w_ref[:], preferred_element_type=jnp.float32)
    y_ref[:] = xl * lax.rsqrt(deg)


def _mlp_body(z0_ref, z1_ref, y_ref, d0_ref, d1_ref, x_ref, bc_ref,
              w1_ref, b1_ref, w2_ref, b2_ref, w3_ref, b3_ref, o_ref):
    deg = d0_ref[:, 0:1] + d1_ref[:, 0:1] + 1.0
    dinv = lax.rsqrt(deg)
    agg = (z0_ref[:] + z1_ref[:] + y_ref[:]) * dinv
    out = jnp.maximum(agg + bc_ref[:], 0.0)
    h = out + x_ref[:]
    h = jnp.maximum(
        jnp.dot(h, w1_ref[:], preferred_element_type=jnp.float32) + b1_ref[:], 0.0)
    h = jnp.maximum(
        jnp.dot(h, w2_ref[:], preferred_element_type=jnp.float32) + b2_ref[:], 0.0)
    o_ref[:] = jnp.dot(h, w3_ref[:], preferred_element_type=jnp.float32) + b3_ref[:]


ROW_BLK = 640


@jax.jit
def kernel(x, edge_index, W_conv, b_conv, W1, b1, W2, b2, W3, b3):
    src = edge_index[0]
    dst = edge_index[1]
    pad = E_PAD - E
    srcp = jnp.concatenate([src, jnp.zeros((pad,), jnp.int32)]).reshape(
        NW * CHUNKS_PER_TILE, CHUNK)
    dstp = jnp.concatenate([dst, jnp.full((pad,), N, jnp.int32)]).reshape(
        NW * CHUNKS_PER_TILE, CHUNK)
    x_pad = jnp.zeros((N_PAD, D), jnp.float32).at[:N].set(x)

    ones_rows = jnp.ones((CHUNK, D), jnp.float32)
    zrows = jnp.zeros((ROWS_PER_TILE, D), jnp.float32)

    deg_parts = _deg_kernel(dstp, zrows, ones_rows)
    deg0, deg1 = deg_parts[0], deg_parts[1]

    y = pl.pallas_call(
        _y_body,
        grid=(N_PAD // ROW_BLK,),
        in_specs=[
            pl.BlockSpec((ROW_BLK, D), lambda i: (i, 0)),
            pl.BlockSpec((D, D), lambda i: (0, 0)),
            pl.BlockSpec((ROW_BLK, D), lambda i: (i, 0)),
            pl.BlockSpec((ROW_BLK, D), lambda i: (i, 0)),
        ],
        out_specs=pl.BlockSpec((ROW_BLK, D), lambda i: (i, 0)),
        out_shape=jax.ShapeDtypeStruct((N_PAD, D), jnp.float32),
    )(x_pad, W_conv, deg0, deg1)

    z_parts = _scatter_kernel(y, srcp, dstp, zrows)
    z0, z1 = z_parts[0], z_parts[1]

    out_pad = pl.pallas_call(
        _mlp_body,
        grid=(N_PAD // ROW_BLK,),
        in_specs=[
            pl.BlockSpec((ROW_BLK, D), lambda i: (i, 0)),
            pl.BlockSpec((ROW_BLK, D), lambda i: (i, 0)),
            pl.BlockSpec((ROW_BLK, D), lambda i: (i, 0)),
            pl.BlockSpec((ROW_BLK, D), lambda i: (i, 0)),
            pl.BlockSpec((ROW_BLK, D), lambda i: (i, 0)),
            pl.BlockSpec((ROW_BLK, D), lambda i: (i, 0)),
            pl.BlockSpec((1, D), lambda i: (0, 0)),
            pl.BlockSpec((D, M), lambda i: (0, 0)),
            pl.BlockSpec((1, M), lambda i: (0, 0)),
            pl.BlockSpec((M, M), lambda i: (0, 0)),
            pl.BlockSpec((1, M), lambda i: (0, 0)),
            pl.BlockSpec((M, 1), lambda i: (0, 0)),
            pl.BlockSpec((1, 1), lambda i: (0, 0)),
        ],
        out_specs=pl.BlockSpec((ROW_BLK, 1), lambda i: (i, 0)),
        out_shape=jax.ShapeDtypeStruct((N_PAD, 1), jnp.float32),
    )(z0, z1, y, deg0, deg1, x_pad,
      b_conv.reshape(1, D), W1, b1.reshape(1, M), W2, b2.reshape(1, M),
      W3, b3.reshape(1, 1))

    return out_pad[:N]
